# Initial kernel scaffold; baseline (speedup 1.0000x reference)
import jax
import jax.numpy as jnp
from jax.experimental import pallas as pl


def kernel(input, target):
    raise NotImplementedError("write your pallas kernel here")


# speedup vs baseline: 10.8354x; 10.8354x over previous
import functools

import jax
import jax.numpy as jnp
from jax import lax
from jax.experimental import pallas as pl
from jax.experimental.pallas import tpu as pltpu


N = 8192
LOGN = 13
ROWS = 128


def _wass_body(x_ref, o_ref, buf):
    buf[...] = x_ref[...]
    idx = lax.broadcasted_iota(jnp.int32, (ROWS, N), 1)

    def stage(x, j, k):
        bit_j = (idx & j) != 0
        partner = jnp.where(bit_j, pltpu.roll(x, j, 1), pltpu.roll(x, N - j, 1))
        mn = jnp.minimum(x, partner)
        mx = jnp.maximum(x, partner)
        take_max = bit_j != ((idx & k) != 0)
        return jnp.where(take_max, mx, mn)

    def outer(k_log, _):
        k = lax.shift_left(jnp.int32(1), k_log)

        def inner(t, _):
            j = lax.shift_right_logical(k, t + 1)
            buf[...] = stage(buf[...], j, k)
            return 0

        lax.fori_loop(0, k_log, inner, 0)
        return 0

    lax.fori_loop(1, LOGN + 1, outer, 0)

    x = buf[...]
    diff = jnp.abs(x[: ROWS // 2, :] - x[ROWS // 2:, :])
    s = jnp.sum(diff, axis=(0, 1), keepdims=True)
    o_ref[...] = s * (1.0 / (N * (ROWS // 2)))


@jax.jit
def kernel(input, target):
    x = jnp.concatenate([input, target], axis=0)
    out = pl.pallas_call(
        _wass_body,
        out_shape=jax.ShapeDtypeStruct((1, 1), jnp.float32),
        scratch_shapes=[pltpu.VMEM((ROWS, N), jnp.float32)],
    )(x)
    return out[0, 0]


# speedup vs baseline: 21.6983x; 2.0025x over previous
import functools

import numpy as np
import jax
import jax.numpy as jnp
from jax import lax
from jax.experimental import pallas as pl
from jax.experimental.pallas import tpu as pltpu
from jax.experimental.pallas import tpu_sc as plsc

N = 8192
L = 16
V = N // L
R = 64
NW = 32
RPW = R // NW
NBINS = 256
HIST = NBINS * L

_I32MIN = np.int32(-2147483648)


def _lane():
    return lax.iota(jnp.int32, 16)


def _to_sortable(f):
    u = lax.bitcast_convert_type(f, jnp.int32)
    mask = lax.shift_right_arithmetic(u, 31) | _I32MIN
    return u ^ mask


def _from_sortable(s):
    mask = lax.shift_right_arithmetic(~s, 31) | _I32MIN
    return lax.bitcast_convert_type(s ^ mask, jnp.float32)


def _sort_row(src_f32, ka, kb, hist):
    def conv_body(i, _):
        f = src_f32[pl.ds(i * L, L)]
        ka[pl.ds(i * L, L)] = _to_sortable(f)
        return 0

    lax.fori_loop(0, V, conv_body, 0)

    ones = jnp.ones((L,), jnp.int32)
    lane = _lane()

    for p in range(4):
        src, dst = (ka, kb) if p % 2 == 0 else (kb, ka)
        shift = 8 * p

        def zero_body(h, _):
            hist[pl.ds(h * L, L)] = jnp.zeros((L,), jnp.int32)
            return 0

        lax.fori_loop(0, NBINS, zero_body, 0)

        def count_body(i, _):
            k = src[pl.ds(i * L, L)]
            d = lax.shift_right_logical(k, shift) & 0xFF
            idx = d * L + lane
            plsc.addupdate_scatter(hist, [idx], ones)
            return 0

        lax.fori_loop(0, V, count_body, 0)

        def scan_body(d, carry):
            hrow = hist[pl.ds(d * L, L)]
            cs = plsc.cumsum(hrow)
            excl = cs - hrow
            carry_v = lax.broadcast_in_dim(carry, (L,), ())
            hist[pl.ds(d * L, L)] = excl + carry_v
            return carry + jnp.sum(hrow)

        lax.fori_loop(0, NBINS, scan_body, jnp.int32(0))

        last = p == 3

        def scatter_body(i, _):
            k = src[pl.ds(i * L, L)]
            d = lax.shift_right_logical(k, shift) & 0xFF
            idx = d * L + lane
            r = plsc.load_gather(hist, [idx])
            plsc.addupdate_scatter(hist, [idx], ones)
            if last:
                pos = r
            else:
                pos = lax.shift_left(r & (V - 1), 4) + lax.shift_right_logical(r, 9)
            plsc.store_scatter(dst, [pos], k)
            return 0

        lax.fori_loop(0, V, scatter_body, 0)


def _body(input_hbm, target_hbm, out_hbm, stage, ua, ub, va, vb, hist, accv):
    wid = lax.axis_index("s") * 2 + lax.axis_index("c")

    def row_body(rr, acc):
        row = wid * RPW + rr
        pltpu.sync_copy(input_hbm.at[row], stage)
        _sort_row(stage, ua, ub, hist)
        pltpu.sync_copy(target_hbm.at[row], stage)
        _sort_row(stage, va, vb, hist)

        def diff_body(i, a):
            fu = _from_sortable(ua[pl.ds(i * L, L)])
            fv = _from_sortable(va[pl.ds(i * L, L)])
            return a + jnp.abs(fu - fv)

        return lax.fori_loop(0, V, diff_body, acc)

    acc = lax.fori_loop(0, RPW, row_body, jnp.zeros((L,), jnp.float32))
    accv[...] = acc
    pltpu.sync_copy(accv, out_hbm.at[wid])


@jax.jit
def kernel(input, target):
    mesh = plsc.VectorSubcoreMesh(
        core_axis_name="c", subcore_axis_name="s", num_cores=2, num_subcores=16
    )
    partials = pl.kernel(
        _body,
        mesh=mesh,
        compiler_params=pltpu.CompilerParams(needs_layout_passes=False),
        out_type=jax.ShapeDtypeStruct((NW, L), jnp.float32),
        scratch_types=[
            pltpu.VMEM((N,), jnp.float32),
            pltpu.VMEM((N,), jnp.int32),
            pltpu.VMEM((N,), jnp.int32),
            pltpu.VMEM((N,), jnp.int32),
            pltpu.VMEM((N,), jnp.int32),
            pltpu.VMEM((HIST,), jnp.int32),
            pltpu.VMEM((L,), jnp.float32),
        ],
    )(input, target)
    return jnp.sum(partials) * (1.0 / (N * R))


# speedup vs baseline: 22.4106x; 1.0328x over previous
import functools

import numpy as np
import jax
import jax.numpy as jnp
from jax import lax
from jax.experimental import pallas as pl
from jax.experimental.pallas import tpu as pltpu
from jax.experimental.pallas import tpu_sc as plsc

N = 8192
L = 16
V = N // L
H = 4
VH = V // H
R = 64
NW = 32
RPW = R // NW
NBINS = 256
TBL = NBINS * L
HIST = H * TBL

_I32MIN = np.int32(-2147483648)

_GDN = lax.GatherDimensionNumbers(
    offset_dims=(), collapsed_slice_dims=(0,), start_index_map=(0,)
)


def _bcast_last(x):
    idx = jnp.full((L, 1), L - 1, jnp.int32)
    return lax.gather(x, idx, _GDN, (1,),
                      mode=lax.GatherScatterMode.PROMISE_IN_BOUNDS)


def _to_sortable(f):
    u = lax.bitcast_convert_type(f, jnp.int32)
    return u ^ (lax.shift_right_arithmetic(u, 31) | _I32MIN)


def _from_sortable(s):
    return lax.bitcast_convert_type(
        s ^ (lax.shift_right_arithmetic(~s, 31) | _I32MIN), jnp.float32)


def _sort_row(src_f32, ka, kb, hist):
    def conv_body(i, _):
        ka[pl.ds(i * L, L)] = _to_sortable(src_f32[pl.ds(i * L, L)])
        return 0

    lax.fori_loop(0, V, conv_body, 0, unroll=4)

    ones = jnp.ones((L,), jnp.int32)
    lane = lax.iota(jnp.int32, L)
    zeros16 = jnp.zeros((L,), jnp.int32)

    for p in range(4):
        src, dst = (ka, kb) if p % 2 == 0 else (kb, ka)
        shift = 8 * p

        def zero_body(d, _):
            for h in range(H):
                hist[pl.ds(h * TBL + d * L, L)] = zeros16
            return 0

        lax.fori_loop(0, NBINS, zero_body, 0, unroll=4)

        def count_body(i, _):
            for h in range(H):
                k = src[pl.ds(h * (VH * L) + i * L, L)]
                d = lax.shift_right_logical(k, shift) & 0xFF
                plsc.addupdate_scatter(hist, [h * TBL + d * L + lane], ones)
            return 0

        lax.fori_loop(0, VH, count_body, 0, unroll=2)

        def scan_body(d, carry_v):
            rows = [hist[pl.ds(h * TBL + d * L, L)] for h in range(H)]
            t = rows[0] + rows[1] + rows[2] + rows[3]
            cs = plsc.cumsum(t)
            start = cs - t + carry_v
            for h in range(H):
                hist[pl.ds(h * TBL + d * L, L)] = start
                start = start + rows[h]
            return carry_v + _bcast_last(cs)

        lax.fori_loop(0, NBINS, scan_body, zeros16, unroll=2)

        last = p == 3

        def scatter_body(i, _):
            for h in range(H):
                k = src[pl.ds(h * (VH * L) + i * L, L)]
                d = lax.shift_right_logical(k, shift) & 0xFF
                idx = h * TBL + d * L + lane
                r = plsc.load_gather(hist, [idx])
                plsc.addupdate_scatter(hist, [idx], ones)
                if last:
                    pos = r
                else:
                    pos = lax.shift_left(r & (V - 1), 4) \
                        + lax.shift_right_logical(r, 9)
                plsc.store_scatter(dst, [pos], k)
            return 0

        lax.fori_loop(0, VH, scatter_body, 0)


def _body(input_hbm, target_hbm, out_hbm, stage, ua, ub, va, vb, hist, accv):
    wid = lax.axis_index("s") * 2 + lax.axis_index("c")

    def row_body(rr, accs):
        row = wid * RPW + rr
        pltpu.sync_copy(input_hbm.at[row], stage)
        _sort_row(stage, ua, ub, hist)
        pltpu.sync_copy(target_hbm.at[row], stage)
        _sort_row(stage, va, vb, hist)

        def diff_body(i, a):
            out = []
            for h in range(H):
                fu = _from_sortable(ua[pl.ds(h * (VH * L) + i * L, L)])
                fv = _from_sortable(va[pl.ds(h * (VH * L) + i * L, L)])
                out.append(a[h] + jnp.abs(fu - fv))
            return tuple(out)

        return lax.fori_loop(0, VH, diff_body, accs, unroll=2)

    z = jnp.zeros((L,), jnp.float32)
    accs = lax.fori_loop(0, RPW, row_body, (z, z, z, z))
    accv[...] = accs[0] + accs[1] + accs[2] + accs[3]
    pltpu.sync_copy(accv, out_hbm.at[wid])


@jax.jit
def kernel(input, target):
    mesh = plsc.VectorSubcoreMesh(
        core_axis_name="c", subcore_axis_name="s", num_cores=2, num_subcores=16
    )
    partials = pl.kernel(
        _body,
        mesh=mesh,
        compiler_params=pltpu.CompilerParams(needs_layout_passes=False),
        out_type=jax.ShapeDtypeStruct((NW, L), jnp.float32),
        scratch_types=[
            pltpu.VMEM((N,), jnp.float32),
            pltpu.VMEM((N,), jnp.int32),
            pltpu.VMEM((N,), jnp.int32),
            pltpu.VMEM((N,), jnp.int32),
            pltpu.VMEM((N,), jnp.int32),
            pltpu.VMEM((HIST,), jnp.int32),
            pltpu.VMEM((L,), jnp.float32),
        ],
    )(input, target)
    return jnp.sum(partials) * (1.0 / (N * R))


# speedup vs baseline: 22.4640x; 1.0024x over previous
import functools

import numpy as np
import jax
import jax.numpy as jnp
from jax import lax
from jax.experimental import pallas as pl
from jax.experimental.pallas import tpu as pltpu
from jax.experimental.pallas import tpu_sc as plsc

N = 8192
L = 16
V = N // L
H = 4
VH = V // H
R = 64
NW = 32
RPW = R // NW
NBINS = 256
TBL = NBINS * L
HIST = H * TBL

_I32MIN = np.int32(-2147483648)

_GDN = lax.GatherDimensionNumbers(
    offset_dims=(), collapsed_slice_dims=(0,), start_index_map=(0,)
)


def _bcast_last(x):
    idx = jnp.full((L, 1), L - 1, jnp.int32)
    return lax.gather(x, idx, _GDN, (1,),
                      mode=lax.GatherScatterMode.PROMISE_IN_BOUNDS)


def _to_sortable(f):
    u = lax.bitcast_convert_type(f, jnp.int32)
    return u ^ (lax.shift_right_arithmetic(u, 31) | _I32MIN)


def _from_sortable(s):
    return lax.bitcast_convert_type(
        s ^ (lax.shift_right_arithmetic(~s, 31) | _I32MIN), jnp.float32)


def _sort_row(src_f32, ka, kb, hists):
    def conv_body(i, _):
        ka[pl.ds(i * L, L)] = _to_sortable(src_f32[pl.ds(i * L, L)])
        return 0

    lax.fori_loop(0, V, conv_body, 0, unroll=4)

    ones = jnp.ones((L,), jnp.int32)
    lane = lax.iota(jnp.int32, L)
    zeros16 = jnp.zeros((L,), jnp.int32)

    for p in range(4):
        src, dst = (ka, kb) if p % 2 == 0 else (kb, ka)
        shift = 8 * p

        def zero_body(d, _):
            for h in range(H):
                hists[h][pl.ds(d * L, L)] = zeros16
            return 0

        lax.fori_loop(0, NBINS, zero_body, 0, unroll=4)

        def count_body(i, _):
            for h in range(H):
                k = src[pl.ds(h * (VH * L) + i * L, L)]
                d = lax.shift_right_logical(k, shift) & 0xFF
                plsc.addupdate_scatter(hists[h], [d * L + lane], ones)
            return 0

        lax.fori_loop(0, VH, count_body, 0, unroll=2)

        def scan_body(d, carry_v):
            rows = [hists[h][pl.ds(d * L, L)] for h in range(H)]
            t = rows[0] + rows[1] + rows[2] + rows[3]
            cs = plsc.cumsum(t)
            start = cs - t + carry_v
            for h in range(H):
                hists[h][pl.ds(d * L, L)] = start
                start = start + rows[h]
            return carry_v + _bcast_last(cs)

        lax.fori_loop(0, NBINS, scan_body, zeros16, unroll=2)

        last = p == 3

        def scatter_body(i, _):
            for h in range(H):
                k = src[pl.ds(h * (VH * L) + i * L, L)]
                d = lax.shift_right_logical(k, shift) & 0xFF
                idx = d * L + lane
                r = plsc.load_gather(hists[h], [idx])
                plsc.addupdate_scatter(hists[h], [idx], ones)
                if last:
                    pos = r
                else:
                    pos = lax.shift_left(r & (V - 1), 4) \
                        + lax.shift_right_logical(r, 9)
                plsc.store_scatter(dst, [pos], k)
            return 0

        lax.fori_loop(0, VH, scatter_body, 0)


def _body(input_hbm, target_hbm, out_hbm, stage, ua, ub, va, vb,
          h0, h1, h2, h3, accv):
    wid = lax.axis_index("s") * 2 + lax.axis_index("c")

    def row_body(rr, accs):
        row = wid * RPW + rr
        pltpu.sync_copy(input_hbm.at[row], stage)
        _sort_row(stage, ua, ub, (h0, h1, h2, h3))
        pltpu.sync_copy(target_hbm.at[row], stage)
        _sort_row(stage, va, vb, (h0, h1, h2, h3))

        def diff_body(i, a):
            out = []
            for h in range(H):
                fu = _from_sortable(ua[pl.ds(h * (VH * L) + i * L, L)])
                fv = _from_sortable(va[pl.ds(h * (VH * L) + i * L, L)])
                out.append(a[h] + jnp.abs(fu - fv))
            return tuple(out)

        return lax.fori_loop(0, VH, diff_body, accs, unroll=2)

    z = jnp.zeros((L,), jnp.float32)
    accs = lax.fori_loop(0, RPW, row_body, (z, z, z, z))
    accv[...] = accs[0] + accs[1] + accs[2] + accs[3]
    pltpu.sync_copy(accv, out_hbm.at[wid])


@jax.jit
def kernel(input, target):
    mesh = plsc.VectorSubcoreMesh(
        core_axis_name="c", subcore_axis_name="s", num_cores=2, num_subcores=16
    )
    partials = pl.kernel(
        _body,
        mesh=mesh,
        compiler_params=pltpu.CompilerParams(needs_layout_passes=False),
        out_type=jax.ShapeDtypeStruct((NW, L), jnp.float32),
        scratch_types=[
            pltpu.VMEM((N,), jnp.float32),
            pltpu.VMEM((N,), jnp.int32),
            pltpu.VMEM((N,), jnp.int32),
            pltpu.VMEM((N,), jnp.int32),
            pltpu.VMEM((N,), jnp.int32),
            pltpu.VMEM((TBL,), jnp.int32),
            pltpu.VMEM((TBL,), jnp.int32),
            pltpu.VMEM((TBL,), jnp.int32),
            pltpu.VMEM((TBL,), jnp.int32),
            pltpu.VMEM((L,), jnp.float32),
        ],
    )(input, target)
    return jnp.sum(partials) * (1.0 / (N * R))


# speedup vs baseline: 39.9224x; 1.7772x over previous
import functools

import numpy as np
import jax
import jax.numpy as jnp
from jax import lax
from jax.experimental import pallas as pl
from jax.experimental.pallas import tpu as pltpu
from jax.experimental.pallas import tpu_sc as plsc

N = 8192
L = 16
V = N // L
H = 4
VH = V // H
R = 64
NW = 32
RPW = R // NW
NBINS = 256
TBL = NBINS * L
HIST = H * TBL

_I32MIN = np.int32(-2147483648)

_GDN = lax.GatherDimensionNumbers(
    offset_dims=(), collapsed_slice_dims=(0,), start_index_map=(0,)
)


def _bcast_last(x):
    idx = jnp.full((L, 1), L - 1, jnp.int32)
    return lax.gather(x, idx, _GDN, (1,),
                      mode=lax.GatherScatterMode.PROMISE_IN_BOUNDS)


def _to_sortable(f):
    u = lax.bitcast_convert_type(f, jnp.int32)
    return u ^ (lax.shift_right_arithmetic(u, 31) | _I32MIN)


def _from_sortable(s):
    return lax.bitcast_convert_type(
        s ^ (lax.shift_right_arithmetic(~s, 31) | _I32MIN), jnp.float32)


def _sort_row(src_f32, ka, kb, hists):
    def conv_body(i, _):
        ka[pl.ds(i * L, L)] = _to_sortable(src_f32[pl.ds(i * L, L)])
        return 0

    lax.fori_loop(0, V, conv_body, 0, unroll=4)

    ones = jnp.ones((L,), jnp.int32)
    lane = lax.iota(jnp.int32, L)
    zeros16 = jnp.zeros((L,), jnp.int32)

    for p in range(4):
        src, dst = (ka, kb) if p % 2 == 0 else (kb, ka)
        shift = 8 * p

        def zero_body(d, _):
            for h in range(H):
                hists[h][pl.ds(d * L, L)] = zeros16
            return 0

        lax.fori_loop(0, NBINS, zero_body, 0, unroll=4)

        def count_body(i, _):
            ks = [src[pl.ds(h * (VH * L) + i * L, L)] for h in range(H)]
            idxs = [(lax.shift_right_logical(k, shift) & 0xFF) * L + lane
                    for k in ks]
            for h in range(H):
                plsc.addupdate_scatter(hists[h], [idxs[h]], ones)
            return 0

        lax.fori_loop(0, VH, count_body, 0, unroll=4)

        def scan_body(d, carry_v):
            rows = [hists[h][pl.ds(d * L, L)] for h in range(H)]
            t = rows[0] + rows[1] + rows[2] + rows[3]
            cs = plsc.cumsum(t)
            start = cs - t + carry_v
            for h in range(H):
                hists[h][pl.ds(d * L, L)] = start
                start = start + rows[h]
            return carry_v + _bcast_last(cs)

        lax.fori_loop(0, NBINS, scan_body, zeros16, unroll=4)

        last = p == 3

        def scatter_body(i, _):
            ks = [src[pl.ds(h * (VH * L) + i * L, L)] for h in range(H)]
            idxs = [(lax.shift_right_logical(k, shift) & 0xFF) * L + lane
                    for k in ks]
            rs = [plsc.load_gather(hists[h], [idxs[h]]) for h in range(H)]
            for h in range(H):
                plsc.addupdate_scatter(hists[h], [idxs[h]], ones)
            if last:
                poss = rs
            else:
                poss = [lax.shift_left(r & (V - 1), 4)
                        + lax.shift_right_logical(r, 9) for r in rs]
            for h in range(H):
                plsc.store_scatter(dst, [poss[h]], ks[h])
            return 0

        lax.fori_loop(0, VH, scatter_body, 0, unroll=2)


def _body(input_hbm, target_hbm, out_hbm, stage, ua, ub, va, vb,
          h0, h1, h2, h3, accv):
    wid = lax.axis_index("s") * 2 + lax.axis_index("c")

    def row_body(rr, accs):
        row = wid * RPW + rr
        pltpu.sync_copy(input_hbm.at[row], stage)
        _sort_row(stage, ua, ub, (h0, h1, h2, h3))
        pltpu.sync_copy(target_hbm.at[row], stage)
        _sort_row(stage, va, vb, (h0, h1, h2, h3))

        def diff_body(i, a):
            out = []
            for h in range(H):
                fu = _from_sortable(ua[pl.ds(h * (VH * L) + i * L, L)])
                fv = _from_sortable(va[pl.ds(h * (VH * L) + i * L, L)])
                out.append(a[h] + jnp.abs(fu - fv))
            return tuple(out)

        return lax.fori_loop(0, VH, diff_body, accs, unroll=2)

    z = jnp.zeros((L,), jnp.float32)
    accs = lax.fori_loop(0, RPW, row_body, (z, z, z, z))
    accv[...] = accs[0] + accs[1] + accs[2] + accs[3]
    pltpu.sync_copy(accv, out_hbm.at[wid])


@jax.jit
def kernel(input, target):
    mesh = plsc.VectorSubcoreMesh(
        core_axis_name="c", subcore_axis_name="s", num_cores=2, num_subcores=16
    )
    partials = pl.kernel(
        _body,
        mesh=mesh,
        compiler_params=pltpu.CompilerParams(needs_layout_passes=False),
        out_type=jax.ShapeDtypeStruct((NW, L), jnp.float32),
        scratch_types=[
            pltpu.VMEM((N,), jnp.float32),
            pltpu.VMEM((N,), jnp.int32),
            pltpu.VMEM((N,), jnp.int32),
            pltpu.VMEM((N,), jnp.int32),
            pltpu.VMEM((N,), jnp.int32),
            pltpu.VMEM((TBL,), jnp.int32),
            pltpu.VMEM((TBL,), jnp.int32),
            pltpu.VMEM((TBL,), jnp.int32),
            pltpu.VMEM((TBL,), jnp.int32),
            pltpu.VMEM((L,), jnp.float32),
        ],
    )(input, target)
    return jnp.sum(partials) * (1.0 / (N * R))


# speedup vs baseline: 58.4962x; 1.4652x over previous
import functools

import numpy as np
import jax
import jax.numpy as jnp
from jax import lax
from jax.experimental import pallas as pl
from jax.experimental.pallas import tpu as pltpu
from jax.experimental.pallas import tpu_sc as plsc

N = 8192
L = 16
V = N // L
H = 4
VH = V // H
R = 64
NW = 32
RPW = R // NW
NBINS = 256
TBL = NBINS * L
HIST = H * TBL

_I32MIN = np.int32(-2147483648)

_GDN = lax.GatherDimensionNumbers(
    offset_dims=(), collapsed_slice_dims=(0,), start_index_map=(0,)
)


def _bcast_last(x):
    idx = jnp.full((L, 1), L - 1, jnp.int32)
    return lax.gather(x, idx, _GDN, (1,),
                      mode=lax.GatherScatterMode.PROMISE_IN_BOUNDS)


def _to_sortable(f):
    u = lax.bitcast_convert_type(f, jnp.int32)
    return u ^ (lax.shift_right_arithmetic(u, 31) | _I32MIN)


def _from_sortable(s):
    return lax.bitcast_convert_type(
        s ^ (lax.shift_right_arithmetic(~s, 31) | _I32MIN), jnp.float32)


def _sort_row(src_f32, ka, kb, hists):
    ones = jnp.ones((L,), jnp.int32)
    lane = lax.iota(jnp.int32, L)
    zeros16 = jnp.zeros((L,), jnp.int32)

    for p in range(4):
        src, dst = (ka, kb) if p % 2 == 0 else (kb, ka)
        shift = 8 * p

        def zero_body(d, _):
            for h in range(H):
                hists[h][pl.ds(d * L, L)] = zeros16
            return 0

        lax.fori_loop(0, NBINS, zero_body, 0, unroll=4)

        if p == 0:
            def count_body(i, _):
                fs = [src_f32[pl.ds(h * (VH * L) + i * L, L)]
                      for h in range(H)]
                ks = [_to_sortable(f) for f in fs]
                for h in range(H):
                    ka[pl.ds(h * (VH * L) + i * L, L)] = ks[h]
                idxs = [(k & 0xFF) * L + lane for k in ks]
                for h in range(H):
                    plsc.addupdate_scatter(hists[h], [idxs[h]], ones)
                return 0
        else:
            def count_body(i, _):
                ks = [src[pl.ds(h * (VH * L) + i * L, L)] for h in range(H)]
                idxs = [(lax.shift_right_logical(k, shift) & 0xFF) * L + lane
                        for k in ks]
                for h in range(H):
                    plsc.addupdate_scatter(hists[h], [idxs[h]], ones)
                return 0

        lax.fori_loop(0, VH, count_body, 0, unroll=4)

        DB = 4

        def scan_body(dd, carry_v):
            base = dd * DB
            rows = [[hists[h][pl.ds((base + q) * L, L)] for h in range(H)]
                    for q in range(DB)]
            ts = [(r[0] + r[1]) + (r[2] + r[3]) for r in rows]
            css = [plsc.cumsum(t) for t in ts]
            bcs = [_bcast_last(cs) for cs in css]
            for q in range(DB):
                start = css[q] - ts[q] + carry_v
                for h in range(H):
                    hists[h][pl.ds((base + q) * L, L)] = start
                    start = start + rows[q][h]
                carry_v = carry_v + bcs[q]
            return carry_v

        lax.fori_loop(0, NBINS // DB, scan_body, zeros16)

        last = p == 3

        def scatter_body(i, _):
            ks = [src[pl.ds(h * (VH * L) + i * L, L)] for h in range(H)]
            idxs = [(lax.shift_right_logical(k, shift) & 0xFF) * L + lane
                    for k in ks]
            rs = [plsc.load_gather(hists[h], [idxs[h]]) for h in range(H)]
            for h in range(H):
                plsc.addupdate_scatter(hists[h], [idxs[h]], ones)
            if last:
                poss = rs
            else:
                poss = [lax.shift_left(r & (V - 1), 4)
                        + lax.shift_right_logical(r, 9) for r in rs]
            for h in range(H):
                plsc.store_scatter(dst, [poss[h]], ks[h])
            return 0

        lax.fori_loop(0, VH, scatter_body, 0, unroll=2)


def _body(input_hbm, target_hbm, out_hbm, stage, ua, ub, va, vb,
          h0, h1, h2, h3, accv):
    wid = lax.axis_index("s") * 2 + lax.axis_index("c")

    def row_body(rr, accs):
        row = wid * RPW + rr
        pltpu.sync_copy(input_hbm.at[row], stage)
        _sort_row(stage, ua, ub, (h0, h1, h2, h3))
        pltpu.sync_copy(target_hbm.at[row], stage)
        _sort_row(stage, va, vb, (h0, h1, h2, h3))

        def diff_body(i, a):
            out = []
            for h in range(H):
                fu = _from_sortable(ua[pl.ds(h * (VH * L) + i * L, L)])
                fv = _from_sortable(va[pl.ds(h * (VH * L) + i * L, L)])
                out.append(a[h] + jnp.abs(fu - fv))
            return tuple(out)

        return lax.fori_loop(0, VH, diff_body, accs, unroll=2)

    z = jnp.zeros((L,), jnp.float32)
    accs = lax.fori_loop(0, RPW, row_body, (z, z, z, z))
    accv[...] = accs[0] + accs[1] + accs[2] + accs[3]
    pltpu.sync_copy(accv, out_hbm.at[wid])


@jax.jit
def kernel(input, target):
    mesh = plsc.VectorSubcoreMesh(
        core_axis_name="c", subcore_axis_name="s", num_cores=2, num_subcores=16
    )
    partials = pl.kernel(
        _body,
        mesh=mesh,
        compiler_params=pltpu.CompilerParams(needs_layout_passes=False),
        out_type=jax.ShapeDtypeStruct((NW, L), jnp.float32),
        scratch_types=[
            pltpu.VMEM((N,), jnp.float32),
            pltpu.VMEM((N,), jnp.int32),
            pltpu.VMEM((N,), jnp.int32),
            pltpu.VMEM((N,), jnp.int32),
            pltpu.VMEM((N,), jnp.int32),
            pltpu.VMEM((TBL,), jnp.int32),
            pltpu.VMEM((TBL,), jnp.int32),
            pltpu.VMEM((TBL,), jnp.int32),
            pltpu.VMEM((TBL,), jnp.int32),
            pltpu.VMEM((L,), jnp.float32),
        ],
    )(input, target)
    return jnp.sum(partials) * (1.0 / (N * R))
